# baseline (speedup 1.0000x reference)
import numpy as np

import jax
import jax.numpy as jnp
from jax.experimental import pallas as pl
from jax.experimental.pallas import tpu as pltpu

N_Q = 8
BINS = 1024
DIM = 256
ROWS_PER_TILE = 1024


def _rvq_body(x_ref, cb_ref, q_out_ref, codes_ref, c2_ref):
    @pl.when(pl.program_id(0) == 0)
    def _():
        cb_all = cb_ref[...]
        c2_ref[...] = jnp.sum(cb_all * cb_all, axis=2)

    x0 = x_ref[...]
    r = x0
    rows = x0.shape[0]
    iota = jax.lax.broadcasted_iota(jnp.int32, (rows, BINS), 1)
    for i in range(N_Q):
        cb = cb_ref[i]
        r2 = jnp.sum(r * r, axis=1, keepdims=True)
        cross = jax.lax.dot_general(
            r, cb, (((1,), (1,)), ((), ())), preferred_element_type=jnp.float32)
        dist = r2 - 2.0 * cross + c2_ref[i][None, :]
        m = jnp.min(dist, axis=1, keepdims=True)
        idx = jnp.min(jnp.where(dist == m, iota, BINS), axis=1)
        codes_ref[i, :] = idx
        onehot = (iota == idx[:, None]).astype(jnp.float32)
        q = jax.lax.dot_general(
            onehot, cb, (((1,), (0,)), ((), ())),
            precision=jax.lax.Precision.HIGHEST,
            preferred_element_type=jnp.float32)
        r = r - q
    q_out_ref[...] = x0 - r


def kernel(x, codebooks, frame_rate):
    b, d, t = x.shape
    n_q, bins, dim = codebooks.shape
    rows = b * t
    xt = jnp.transpose(x, (0, 2, 1)).reshape(rows, dim)

    grid = (rows // ROWS_PER_TILE,)
    q2d, codes2d = pl.pallas_call(
        _rvq_body,
        grid=grid,
        in_specs=[
            pl.BlockSpec((ROWS_PER_TILE, dim), lambda i: (i, 0)),
            pl.BlockSpec((n_q, bins, dim), lambda i: (0, 0, 0)),
        ],
        out_specs=[
            pl.BlockSpec((ROWS_PER_TILE, dim), lambda i: (i, 0)),
            pl.BlockSpec((n_q, ROWS_PER_TILE), lambda i: (0, i)),
        ],
        out_shape=[
            jax.ShapeDtypeStruct((rows, dim), jnp.float32),
            jax.ShapeDtypeStruct((n_q, rows), jnp.int32),
        ],
        scratch_shapes=[pltpu.VMEM((n_q, bins), jnp.float32)],
        compiler_params=pltpu.CompilerParams(
            dimension_semantics=("arbitrary",)),
    )(xt, codebooks)

    quantized = jnp.transpose(q2d.reshape(b, t, d), (0, 2, 1))
    codes = codes2d.reshape(n_q, b, t)
    bw = jnp.asarray(n_q * np.log2(bins) * frame_rate, dtype=x.dtype)
    return quantized, codes, bw


# speedup vs baseline: 1.0014x; 1.0014x over previous
import numpy as np

import jax
import jax.numpy as jnp
from jax.experimental import pallas as pl
from jax.experimental.pallas import tpu as pltpu

N_Q = 8
BINS = 1024
DIM = 256
ROWS_PER_TILE = 1024


def _rvq_body(x_ref, cb_ref, q_out_ref, codes_ref, c2_ref):
    @pl.when(pl.program_id(0) == 0)
    def _():
        cb_all = cb_ref[...]
        c2_ref[...] = 0.5 * jnp.sum(cb_all * cb_all, axis=2)

    x0 = x_ref[...]
    r = x0
    rows = x0.shape[0]
    iota = jax.lax.broadcasted_iota(jnp.int32, (rows, BINS), 1)
    for i in range(N_Q):
        cb = cb_ref[i]
        cross = jax.lax.dot_general(
            r, cb, (((1,), (1,)), ((), ())), preferred_element_type=jnp.float32)
        score = cross - c2_ref[i][None, :]
        m = jnp.max(score, axis=1, keepdims=True)
        idx = jnp.min(jnp.where(score == m, iota, BINS), axis=1)
        codes_ref[i, :] = idx
        onehot = (iota == idx[:, None]).astype(jnp.float32)
        q = jax.lax.dot_general(
            onehot, cb, (((1,), (0,)), ((), ())),
            precision=jax.lax.Precision.HIGHEST,
            preferred_element_type=jnp.float32)
        r = r - q
    q_out_ref[...] = x0 - r


def kernel(x, codebooks, frame_rate):
    b, d, t = x.shape
    n_q, bins, dim = codebooks.shape
    rows = b * t
    xt = jnp.transpose(x, (0, 2, 1)).reshape(rows, dim)

    grid = (rows // ROWS_PER_TILE,)
    q2d, codes2d = pl.pallas_call(
        _rvq_body,
        grid=grid,
        in_specs=[
            pl.BlockSpec((ROWS_PER_TILE, dim), lambda i: (i, 0)),
            pl.BlockSpec((n_q, bins, dim), lambda i: (0, 0, 0)),
        ],
        out_specs=[
            pl.BlockSpec((ROWS_PER_TILE, dim), lambda i: (i, 0)),
            pl.BlockSpec((n_q, ROWS_PER_TILE), lambda i: (0, i)),
        ],
        out_shape=[
            jax.ShapeDtypeStruct((rows, dim), jnp.float32),
            jax.ShapeDtypeStruct((n_q, rows), jnp.int32),
        ],
        scratch_shapes=[pltpu.VMEM((n_q, bins), jnp.float32)],
        compiler_params=pltpu.CompilerParams(
            dimension_semantics=("arbitrary",)),
    )(xt, codebooks)

    quantized = jnp.transpose(q2d.reshape(b, t, d), (0, 2, 1))
    codes = codes2d.reshape(n_q, b, t)
    bw = jnp.asarray(n_q * np.log2(bins) * frame_rate, dtype=x.dtype)
    return quantized, codes, bw


# speedup vs baseline: 1.8649x; 1.8623x over previous
import numpy as np

import jax
import jax.numpy as jnp
from jax.experimental import pallas as pl
from jax.experimental.pallas import tpu as pltpu

N_Q = 8
BINS = 1024
DIM = 256
ROWS_PER_TILE = 1024


def _rvq_body(x_ref, cb_ref, q_out_ref, codes_ref, c2_ref, cbh_ref, cbm_ref,
              cbl_ref):
    @pl.when(pl.program_id(0) == 0)
    def _():
        cb_all = cb_ref[...]
        c2_ref[...] = 0.5 * jnp.sum(cb_all * cb_all, axis=2)
        hi = cb_all.astype(jnp.bfloat16)
        r1 = cb_all - hi.astype(jnp.float32)
        mid = r1.astype(jnp.bfloat16)
        cbh_ref[...] = hi
        cbm_ref[...] = mid
        cbl_ref[...] = (r1 - mid.astype(jnp.float32)).astype(jnp.bfloat16)

    x0 = x_ref[...]
    r = x0
    rows = x0.shape[0]
    iota = jax.lax.broadcasted_iota(jnp.int32, (rows, BINS), 1)
    for i in range(N_Q):
        cb = cb_ref[i]
        cross = jax.lax.dot_general(
            r, cb, (((1,), (1,)), ((), ())), preferred_element_type=jnp.float32)
        score = cross - c2_ref[i][None, :]
        m = jnp.max(score, axis=1, keepdims=True)
        idx = jnp.min(jnp.where(score == m, iota, BINS), axis=1)
        codes_ref[i, :] = idx
        onehot = (iota == idx[:, None]).astype(jnp.bfloat16)
        dn = (((1,), (0,)), ((), ()))
        q = ((jax.lax.dot_general(onehot, cbh_ref[i], dn,
                                  preferred_element_type=jnp.float32)
              + jax.lax.dot_general(onehot, cbm_ref[i], dn,
                                    preferred_element_type=jnp.float32))
             + jax.lax.dot_general(onehot, cbl_ref[i], dn,
                                   preferred_element_type=jnp.float32))
        r = r - q
    q_out_ref[...] = x0 - r


def kernel(x, codebooks, frame_rate):
    b, d, t = x.shape
    n_q, bins, dim = codebooks.shape
    rows = b * t
    xt = jnp.transpose(x, (0, 2, 1)).reshape(rows, dim)

    grid = (rows // ROWS_PER_TILE,)
    q2d, codes2d = pl.pallas_call(
        _rvq_body,
        grid=grid,
        in_specs=[
            pl.BlockSpec((ROWS_PER_TILE, dim), lambda i: (i, 0)),
            pl.BlockSpec((n_q, bins, dim), lambda i: (0, 0, 0)),
        ],
        out_specs=[
            pl.BlockSpec((ROWS_PER_TILE, dim), lambda i: (i, 0)),
            pl.BlockSpec((n_q, ROWS_PER_TILE), lambda i: (0, i)),
        ],
        out_shape=[
            jax.ShapeDtypeStruct((rows, dim), jnp.float32),
            jax.ShapeDtypeStruct((n_q, rows), jnp.int32),
        ],
        scratch_shapes=[
            pltpu.VMEM((n_q, bins), jnp.float32),
            pltpu.VMEM((n_q, bins, dim), jnp.bfloat16),
            pltpu.VMEM((n_q, bins, dim), jnp.bfloat16),
            pltpu.VMEM((n_q, bins, dim), jnp.bfloat16),
        ],
        compiler_params=pltpu.CompilerParams(
            dimension_semantics=("arbitrary",)),
    )(xt, codebooks)

    quantized = jnp.transpose(q2d.reshape(b, t, d), (0, 2, 1))
    codes = codes2d.reshape(n_q, b, t)
    bw = jnp.asarray(n_q * np.log2(bins) * frame_rate, dtype=x.dtype)
    return quantized, codes, bw


# speedup vs baseline: 2.6573x; 1.4249x over previous
import numpy as np

import jax
import jax.numpy as jnp
from jax.experimental import pallas as pl
from jax.experimental.pallas import tpu as pltpu

N_Q = 8
BINS = 1024
DIM = 256
ROWS_PER_TILE = 1024


def _rvq_body(x_ref, cb_ref, q_out_ref, codes_ref, c2_ref, cbh_ref, cbm_ref,
              cbl_ref):
    @pl.when(pl.program_id(0) == 0)
    def _():
        cb_all = cb_ref[...]
        c2_ref[...] = 0.5 * jnp.sum(cb_all * cb_all, axis=2)
        hi = cb_all.astype(jnp.bfloat16)
        r1 = cb_all - hi.astype(jnp.float32)
        mid = r1.astype(jnp.bfloat16)
        cbh_ref[...] = hi
        cbm_ref[...] = mid
        cbl_ref[...] = (r1 - mid.astype(jnp.float32)).astype(jnp.bfloat16)

    x0 = x_ref[...]
    rows = x0.shape[0]
    half = rows // 2
    iota = jax.lax.broadcasted_iota(jnp.int32, (half, BINS), 1)
    dn_t = (((1,), (1,)), ((), ()))
    dn = (((1,), (0,)), ((), ()))
    rs = [x0[:half], x0[half:]]
    for i in range(N_Q):
        cb = cb_ref[i]
        for h in range(2):
            cross = jax.lax.dot_general(
                rs[h], cb, dn_t, preferred_element_type=jnp.float32)
            score = cross - c2_ref[i][None, :]
            m = jnp.max(score, axis=1, keepdims=True)
            idx = jnp.min(jnp.where(score == m, iota, BINS), axis=1)
            codes_ref[i, pl.ds(h * half, half)] = idx
            onehot = (iota == idx[:, None]).astype(jnp.bfloat16)
            q = ((jax.lax.dot_general(onehot, cbh_ref[i], dn,
                                      preferred_element_type=jnp.float32)
                  + jax.lax.dot_general(onehot, cbm_ref[i], dn,
                                        preferred_element_type=jnp.float32))
                 + jax.lax.dot_general(onehot, cbl_ref[i], dn,
                                       preferred_element_type=jnp.float32))
            rs[h] = rs[h] - q
    q_out_ref[...] = x0 - jnp.concatenate(rs, axis=0)


def kernel(x, codebooks, frame_rate):
    b, d, t = x.shape
    n_q, bins, dim = codebooks.shape
    rows = b * t
    xt = jnp.transpose(x, (0, 2, 1)).reshape(rows, dim)

    grid = (rows // ROWS_PER_TILE,)
    q2d, codes2d = pl.pallas_call(
        _rvq_body,
        grid=grid,
        in_specs=[
            pl.BlockSpec((ROWS_PER_TILE, dim), lambda i: (i, 0)),
            pl.BlockSpec((n_q, bins, dim), lambda i: (0, 0, 0)),
        ],
        out_specs=[
            pl.BlockSpec((ROWS_PER_TILE, dim), lambda i: (i, 0)),
            pl.BlockSpec((n_q, ROWS_PER_TILE), lambda i: (0, i)),
        ],
        out_shape=[
            jax.ShapeDtypeStruct((rows, dim), jnp.float32),
            jax.ShapeDtypeStruct((n_q, rows), jnp.int32),
        ],
        scratch_shapes=[
            pltpu.VMEM((n_q, bins), jnp.float32),
            pltpu.VMEM((n_q, bins, dim), jnp.bfloat16),
            pltpu.VMEM((n_q, bins, dim), jnp.bfloat16),
            pltpu.VMEM((n_q, bins, dim), jnp.bfloat16),
        ],
        compiler_params=pltpu.CompilerParams(
            dimension_semantics=("arbitrary",)),
    )(xt, codebooks)

    quantized = jnp.transpose(q2d.reshape(b, t, d), (0, 2, 1))
    codes = codes2d.reshape(n_q, b, t)
    bw = jnp.asarray(n_q * np.log2(bins) * frame_rate, dtype=x.dtype)
    return quantized, codes, bw
